# baseline (speedup 1.0000x reference)
import functools

import jax
import jax.numpy as jnp
from jax import lax
from jax.experimental import pallas as pl
from jax.experimental.pallas import tpu as pltpu
from jax.experimental.pallas import tpu_sc as plsc

_CHUNK_ROWS = 8
_NBUF = 4


def kernel(x, emb_weight):
    sl = x.shape[1]
    dim = emb_weight.shape[1]
    info = plsc.get_sparse_core_info()
    nw = info.num_cores * info.num_subcores
    rows_per_w = sl // nw
    nchunks = rows_per_w // _CHUNK_ROWS
    mesh = plsc.VectorSubcoreMesh(core_axis_name="c", subcore_axis_name="s")

    @functools.partial(
        pl.kernel,
        mesh=mesh,
        out_type=jax.ShapeDtypeStruct((sl, dim), emb_weight.dtype),
        scratch_types=(
            [pltpu.VMEM((_NBUF, _CHUNK_ROWS, dim), emb_weight.dtype)]
            + [pltpu.SemaphoreType.DMA] * (2 * _NBUF)
        ),
    )
    def copy_kernel(table_hbm, out_hbm, buf, *sems):
        wid = lax.axis_index("s") * info.num_cores + lax.axis_index("c")
        base = wid * rows_per_w
        in_sems = sems[:_NBUF]
        out_sems = sems[_NBUF:]

        def in_copy(c, b):
            return pltpu.make_async_copy(
                table_hbm.at[pl.ds(base + c * _CHUNK_ROWS, _CHUNK_ROWS)],
                buf.at[b],
                in_sems[b],
            )

        def out_copy(c, b):
            return pltpu.make_async_copy(
                buf.at[b],
                out_hbm.at[pl.ds(base + c * _CHUNK_ROWS, _CHUNK_ROWS)],
                out_sems[b],
            )

        for b in range(_NBUF):
            in_copy(b, b).start()
        for c in range(nchunks):
            b = c % _NBUF
            in_copy(c, b).wait()
            out_copy(c, b).start()
            nxt = c + _NBUF
            if nxt < nchunks:
                out_copy(c, b).wait()
                in_copy(nxt, b).start()
        for c in range(nchunks - _NBUF, nchunks):
            out_copy(c, c % _NBUF).wait()

    return copy_kernel(emb_weight)


# speedup vs baseline: 1.5977x; 1.5977x over previous
import jax
import jax.numpy as jnp
from jax.experimental import pallas as pl


def _copy_body(in_ref, out_ref):
    out_ref[...] = in_ref[...]


def kernel(x, emb_weight):
    sl = x.shape[1]
    dim = emb_weight.shape[1]
    block_rows = 1024
    grid = (sl // block_rows,)
    return pl.pallas_call(
        _copy_body,
        out_shape=jax.ShapeDtypeStruct((sl, dim), emb_weight.dtype),
        grid=grid,
        in_specs=[pl.BlockSpec((block_rows, dim), lambda i: (i, 0))],
        out_specs=pl.BlockSpec((block_rows, dim), lambda i: (i, 0)),
    )(emb_weight)
